# baseline (speedup 1.0000x reference)
import functools
import jax
import jax.numpy as jnp
from jax import lax
from jax.experimental import pallas as pl
from jax.experimental.pallas import tpu as pltpu
from jax.experimental.pallas import tpu_sc as plsc

_NUM_USERS = 1_000_000
_NUM_SUBGROUPS = 1_000
_NUM_CHOICES = 100_000
_D = 64

_NW = 32
_BUF = 1_000
_N_CHUNKS = _NUM_USERS // _BUF
_FULL_ROUNDS = _N_CHUNKS // _NW
_TAIL = _N_CHUNKS - _FULL_ROUNDS * _NW

_CHO_WORKERS = 25
_CPW = _NUM_CHOICES // _CHO_WORKERS
_SUB_WORKERS = 5
_SPW = _NUM_SUBGROUPS // _SUB_WORKERS


def _sc_body(sub_hbm, cho_hbm, user_hbm,
             sub_out, cho_out, user_out,
             buf, sem_u, sem_c, sem_s):
    wid = lax.axis_index("c") * 16 + lax.axis_index("s")

    cbase = wid * _CPW
    cho_copy = pltpu.make_async_copy(
        cho_hbm.at[pl.ds(cbase, _CPW), :],
        cho_out.at[pl.ds(cbase, _CPW), :],
        sem_c)

    @pl.when(wid < _CHO_WORKERS)
    def _():
        cho_copy.start()

    sbase = (wid - _CHO_WORKERS) * _SPW
    sub_copy = pltpu.make_async_copy(
        sub_hbm.at[pl.ds(sbase, _SPW), :],
        sub_out.at[pl.ds(sbase, _SPW), :],
        sem_s)

    @pl.when((wid >= _CHO_WORKERS) & (wid < _CHO_WORKERS + _SUB_WORKERS))
    def _():
        sub_copy.start()

    pltpu.sync_copy(user_hbm, buf.at[pl.ds(0, 1), :])
    regs = [buf[0, pl.ds(k * 16, 16)] for k in range(_D // 16)]

    def _fill_row(j, carry):
        for k in range(_D // 16):
            buf[j, pl.ds(k * 16, 16)] = regs[k]
        return carry

    lax.fori_loop(1, _BUF, _fill_row, 0)

    user_copies = []
    for j in range(_FULL_ROUNDS):
        c = pltpu.make_async_copy(
            buf,
            user_out.at[pl.ds((j * _NW) * _BUF + wid * _BUF, _BUF), :],
            sem_u)
        c.start()
        user_copies.append(c)

    tail_copy = pltpu.make_async_copy(
        buf,
        user_out.at[pl.ds((_FULL_ROUNDS * _NW) * _BUF + wid * _BUF, _BUF), :],
        sem_u)

    @pl.when(wid < _TAIL)
    def _():
        tail_copy.start()

    for c in user_copies:
        c.wait()

    @pl.when(wid < _TAIL)
    def _():
        tail_copy.wait()

    @pl.when(wid < _CHO_WORKERS)
    def _():
        cho_copy.wait()

    @pl.when((wid >= _CHO_WORKERS) & (wid < _CHO_WORKERS + _SUB_WORKERS))
    def _():
        sub_copy.wait()


@functools.partial(
    pl.kernel,
    out_type=[
        jax.ShapeDtypeStruct((_NUM_SUBGROUPS, _D), jnp.float32),
        jax.ShapeDtypeStruct((_NUM_CHOICES, _D), jnp.float32),
        jax.ShapeDtypeStruct((_NUM_USERS, _D), jnp.float32),
    ],
    mesh=plsc.VectorSubcoreMesh(core_axis_name="c", subcore_axis_name="s"),
    scratch_types=[
        pltpu.VMEM((_BUF, _D), jnp.float32),
        pltpu.SemaphoreType.DMA,
        pltpu.SemaphoreType.DMA,
        pltpu.SemaphoreType.DMA,
    ],
)
def _sc_tables(sub_hbm, cho_hbm, user_hbm, sub_out, cho_out, user_out,
               buf, sem_u, sem_c, sem_s):
    _sc_body(sub_hbm, cho_hbm, user_hbm, sub_out, cho_out, user_out,
             buf, sem_u, sem_c, sem_s)


def kernel(sub_w, cho_w, user_token):
    sub_o, cho_o, user_o = _sc_tables(sub_w, cho_w, user_token)
    return (sub_o, cho_o, user_o)


# speedup vs baseline: 2.4544x; 2.4544x over previous
import jax
import jax.numpy as jnp
from jax import lax
from jax.experimental import pallas as pl
from jax.experimental.pallas import tpu as pltpu

_NUM_USERS = 1_000_000
_NUM_SUBGROUPS = 1_000
_NUM_CHOICES = 100_000
_D = 64

_U = _NUM_USERS * _D
_C = _NUM_CHOICES * _D
_S = _NUM_SUBGROUPS * _D

_PAT = 1024
_BLK = 512_000
_N_USER = _U // _BLK
_N_CHO = 8
_CCH = _C // _N_CHO


def _tables_kernel(pat_ref, sub_hbm, cho_hbm,
                   sub_out, cho_out, user_out,
                   scratch, cho_vmem, sub_vmem,
                   sem_u, sem_cr, sem_cw, sem_s):
    cho_reads = []
    for j in range(_N_CHO):
        c = pltpu.make_async_copy(
            cho_hbm.at[pl.ds(j * _CCH, _CCH)],
            cho_vmem.at[pl.ds(j * _CCH, _CCH)],
            sem_cr.at[j])
        c.start()
        cho_reads.append(c)
    sub_read = pltpu.make_async_copy(sub_hbm, sub_vmem, sem_s)
    sub_read.start()

    scratch[pl.ds(0, _PAT)] = pat_ref[...]
    first = scratch[pl.ds(0, _PAT)]

    def _fill(j, carry):
        scratch[pl.ds(j * _PAT, _PAT)] = first
        return carry

    lax.fori_loop(1, _BLK // _PAT, _fill, 0)

    user_copies = []
    for i in range(_N_USER):
        c = pltpu.make_async_copy(
            scratch,
            user_out.at[pl.ds(i * _BLK, _BLK)],
            sem_u)
        c.start()
        user_copies.append(c)

    cho_writes = []
    for j in range(_N_CHO):
        cho_reads[j].wait()
        c = pltpu.make_async_copy(
            cho_vmem.at[pl.ds(j * _CCH, _CCH)],
            cho_out.at[pl.ds(j * _CCH, _CCH)],
            sem_cw)
        c.start()
        cho_writes.append(c)

    sub_read.wait()
    sub_write = pltpu.make_async_copy(sub_vmem, sub_out, sem_s)
    sub_write.start()

    for c in user_copies:
        c.wait()
    for c in cho_writes:
        c.wait()
    sub_write.wait()


def kernel(sub_w, cho_w, user_token):
    tok = user_token.reshape(_D)
    pattern = jnp.tile(tok, _PAT // _D)

    sub_o, cho_o, user_o = pl.pallas_call(
        _tables_kernel,
        in_specs=[
            pl.BlockSpec(memory_space=pltpu.MemorySpace.VMEM),
            pl.BlockSpec(memory_space=pltpu.MemorySpace.HBM),
            pl.BlockSpec(memory_space=pltpu.MemorySpace.HBM),
        ],
        out_specs=[
            pl.BlockSpec(memory_space=pltpu.MemorySpace.HBM),
            pl.BlockSpec(memory_space=pltpu.MemorySpace.HBM),
            pl.BlockSpec(memory_space=pltpu.MemorySpace.HBM),
        ],
        out_shape=[
            jax.ShapeDtypeStruct((_S,), jnp.float32),
            jax.ShapeDtypeStruct((_C,), jnp.float32),
            jax.ShapeDtypeStruct((_U,), jnp.float32),
        ],
        scratch_shapes=[
            pltpu.VMEM((_BLK,), jnp.float32),
            pltpu.VMEM((_C,), jnp.float32),
            pltpu.VMEM((_S,), jnp.float32),
            pltpu.SemaphoreType.DMA,
            pltpu.SemaphoreType.DMA((_N_CHO,)),
            pltpu.SemaphoreType.DMA,
            pltpu.SemaphoreType.DMA,
        ],
    )(pattern, sub_w.reshape(_S), cho_w.reshape(_C))

    return (
        sub_o.reshape(_NUM_SUBGROUPS, _D),
        cho_o.reshape(_NUM_CHOICES, _D),
        user_o.reshape(_NUM_USERS, _D),
    )


# speedup vs baseline: 3.8906x; 1.5852x over previous
import jax
import jax.numpy as jnp
from jax import lax
from jax.experimental import pallas as pl
from jax.experimental.pallas import tpu as pltpu

_NUM_USERS = 1_000_000
_NUM_SUBGROUPS = 1_000
_NUM_CHOICES = 100_000
_D = 64

_SCR_ROWS = 8_000
_N_USER = _NUM_USERS // _SCR_ROWS


def _bcast_kernel(user_ref, user_out, scratch, sem_u):
    scratch[...] = jnp.broadcast_to(user_ref[...], (_SCR_ROWS, _D))
    copies = []
    for i in range(_N_USER):
        c = pltpu.make_async_copy(
            scratch,
            user_out.at[pl.ds(i * _SCR_ROWS, _SCR_ROWS), :],
            sem_u)
        c.start()
        copies.append(c)
    for c in copies:
        c.wait()


def kernel(sub_w, cho_w, user_token):
    user_o = pl.pallas_call(
        _bcast_kernel,
        in_specs=[pl.BlockSpec(memory_space=pltpu.MemorySpace.VMEM)],
        out_specs=pl.BlockSpec(memory_space=pltpu.MemorySpace.HBM),
        out_shape=jax.ShapeDtypeStruct((_NUM_USERS, _D), jnp.float32),
        scratch_shapes=[
            pltpu.VMEM((_SCR_ROWS, _D), jnp.float32),
            pltpu.SemaphoreType.DMA,
        ],
    )(user_token)
    return (sub_w, cho_w, user_o)


# speedup vs baseline: 3.9016x; 1.0028x over previous
import jax
import jax.numpy as jnp
from jax import lax
from jax.experimental import pallas as pl
from jax.experimental.pallas import tpu as pltpu

_NUM_USERS = 1_000_000
_NUM_SUBGROUPS = 1_000
_NUM_CHOICES = 100_000
_D = 64

_SCR_ROWS = 8_000
_N_USER = _NUM_USERS // _SCR_ROWS
_NBUF = 8


def _bcast_kernel(user_ref, user_out, *rest):
    bufs = rest[:_NBUF]
    sems = rest[_NBUF]
    for b in bufs:
        b[...] = jnp.broadcast_to(user_ref[...], (_SCR_ROWS, _D))
    copies = []
    for i in range(_N_USER):
        c = pltpu.make_async_copy(
            bufs[i % _NBUF],
            user_out.at[pl.ds(i * _SCR_ROWS, _SCR_ROWS), :],
            sems.at[i % _NBUF])
        c.start()
        copies.append(c)
    for c in copies:
        c.wait()


def kernel(sub_w, cho_w, user_token):
    user_o = pl.pallas_call(
        _bcast_kernel,
        in_specs=[pl.BlockSpec(memory_space=pltpu.MemorySpace.VMEM)],
        out_specs=pl.BlockSpec(memory_space=pltpu.MemorySpace.HBM),
        out_shape=jax.ShapeDtypeStruct((_NUM_USERS, _D), jnp.float32),
        scratch_shapes=(
            [pltpu.VMEM((_SCR_ROWS, _D), jnp.float32)] * _NBUF
            + [pltpu.SemaphoreType.DMA((_NBUF,))]
        ),
    )(user_token)
    return (sub_w, cho_w, user_o)
